# baseline (speedup 1.0000x reference)
import functools

import jax
import jax.numpy as jnp
from jax import lax
from jax.experimental import pallas as pl
from jax.experimental.pallas import tpu as pltpu
from jax.experimental.pallas import tpu_sc as plsc

_BATCH, _SEQ = 16384, 50
_NROWS = 1000000
_D = 64
_NC, _NS = 2, 16
_NW = _NC * _NS
_NTB = _BATCH // 128
_TBPW = _NTB // _NW
_NCHUNK = _SEQ * _TBPW

_mesh = plsc.VectorSubcoreMesh(core_axis_name="c", subcore_axis_name="s")


@functools.partial(
    pl.kernel,
    mesh=_mesh,
    out_type=jax.ShapeDtypeStruct((_SEQ, _D // 8, 128, 8, 128), jnp.float32),
    scratch_types=[
        pltpu.VMEM((_SEQ, 128 * _TBPW), jnp.int32),
        [pltpu.VMEM((128, _D), jnp.float32) for _ in range(2)],
        [pltpu.VMEM((_D // 8, 8, 129), jnp.float32) for _ in range(2)],
        [pltpu.SemaphoreType.DMA for _ in range(2)],
        [pltpu.SemaphoreType.DMA for _ in range(2)],
    ],
    compiler_params=pltpu.CompilerParams(
        use_tc_tiling_on_sc=False, needs_layout_passes=False),
)
def _embed(xt_hbm, table_hbm, out_hbm, idx_v, gbuf, tbuf, gsem, ssem):
    wid = lax.axis_index("s") * _NC + lax.axis_index("c")

    pltpu.sync_copy(xt_hbm.at[:, pl.ds(wid * 128 * _TBPW, 128 * _TBPW)], idx_v)

    def chunk_su(t):
        return t // _TBPW, t % _TBPW

    def gather_desc(t, p):
        s, jj = chunk_su(t)
        return pltpu.make_async_copy(
            table_hbm.at[idx_v.at[s, pl.ds(jj * 128, 128)]], gbuf[p],
            gsem[p])

    def store_descs(t, p):
        s, jj = chunk_su(t)
        tb = wid * _TBPW + jj
        return [
            pltpu.make_async_copy(
                tbuf[p].at[:, :, pl.ds(0, 128)],
                out_hbm.at[s, :, tb], ssem[p])
        ]

    iota16 = lax.iota(jnp.int32, 16)
    tdvecs = [(iota16 + 16 * k) // 8 for k in range(4)]
    ddvecs = [lax.rem(iota16 + 16 * k, 8) for k in range(4)]

    gather_desc(0, 0).start()

    def step(g, carry):
        def run(t, p):
            gather_desc(t, p).wait()

            @pl.when(t + 1 < _NCHUNK)
            def _():
                gather_desc(t + 1, 1 - p).start()

            prev = None
            for r2 in range(0, 128, 2):
                vs = [gbuf[p][r2 + (kk // 4), pl.ds(16 * (kk % 4), 16)]
                      for kk in range(8)]
                if prev is not None:
                    pr2, pvs = prev
                    for kk in range(8):
                        plsc.store_scatter(
                            tbuf[p],
                            [tdvecs[kk % 4], ddvecs[kk % 4],
                             jnp.full((16,), pr2 + (kk // 4), jnp.int32)],
                            pvs[kk])
                prev = (r2, vs)
            pr2, pvs = prev
            for kk in range(8):
                plsc.store_scatter(
                    tbuf[p],
                    [tdvecs[kk % 4], ddvecs[kk % 4],
                     jnp.full((16,), pr2 + (kk // 4), jnp.int32)],
                    pvs[kk])

            @pl.when(t >= 2)
            def _():
                for d in store_descs(t - 2, p):
                    d.wait()

            for d in store_descs(t, p):
                d.start()

        run(2 * g, 0)
        run(2 * g + 1, 1)
        return carry

    lax.fori_loop(0, _NCHUNK // 2, step, 0)
    for d in store_descs(_NCHUNK - 2, 0):
        d.wait()
    for d in store_descs(_NCHUNK - 1, 1):
        d.wait()


def kernel(x, table):
    xt2 = (x.T * 2).astype(jnp.int32)
    tpad = jnp.pad(table, ((0, 0), (0, _D))).reshape(2 * _NROWS, _D)
    out5 = _embed(xt2, tpad)
    return out5.transpose(2, 4, 0, 1, 3).reshape(_BATCH, _SEQ, _D)


# speedup vs baseline: 1.1168x; 1.1168x over previous
import functools

import jax
import jax.numpy as jnp
from jax import lax
from jax.experimental import pallas as pl
from jax.experimental.pallas import tpu as pltpu
from jax.experimental.pallas import tpu_sc as plsc

_BATCH, _SEQ = 16384, 50
_NROWS = 1000000
_D = 64
_NC, _NS = 2, 16
_NW = _NC * _NS
_NTB = _BATCH // 128
_TBPW = _NTB // _NW
_NCHUNK = _SEQ * _TBPW

_mesh = plsc.VectorSubcoreMesh(core_axis_name="c", subcore_axis_name="s")


@functools.partial(
    pl.kernel,
    mesh=_mesh,
    out_type=jax.ShapeDtypeStruct((_SEQ, _D // 8, 128, 8, 128), jnp.float32),
    scratch_types=[
        pltpu.VMEM((_SEQ, 128 * _TBPW), jnp.int32),
        [pltpu.VMEM((128, _D), jnp.float32) for _ in range(2)],
        [pltpu.VMEM((_D // 8, 8, 129), jnp.float32) for _ in range(2)],
        [pltpu.SemaphoreType.DMA for _ in range(2)],
        [pltpu.SemaphoreType.DMA for _ in range(2)],
    ],
    compiler_params=pltpu.CompilerParams(
        use_tc_tiling_on_sc=False, needs_layout_passes=False),
)
def _embed(xt_hbm, table_hbm, out_hbm, idx_v, gbuf, tbuf, gsem, ssem):
    wid = lax.axis_index("s") * _NC + lax.axis_index("c")

    pltpu.sync_copy(xt_hbm.at[:, pl.ds(wid * 128 * _TBPW, 128 * _TBPW)], idx_v)

    def chunk_su(t):
        return t // _TBPW, t % _TBPW

    def gather_desc(t, p):
        s, jj = chunk_su(t)
        return pltpu.make_async_copy(
            table_hbm.at[idx_v.at[s, pl.ds(jj * 128, 128)]], gbuf[p],
            gsem[p])

    def store_descs(t, p):
        s, jj = chunk_su(t)
        tb = wid * _TBPW + jj
        return [
            pltpu.make_async_copy(
                tbuf[p].at[:, :, pl.ds(0, 128)],
                out_hbm.at[s, :, tb], ssem[p])
        ]

    iota16 = lax.iota(jnp.int32, 16)
    tdvecs = [(iota16 + 16 * k) // 8 for k in range(4)]
    ddvecs = [lax.rem(iota16 + 16 * k, 8) for k in range(4)]

    gather_desc(0, 0).start()

    def step(t, carry):
        p = lax.rem(t, 2)

        def run(p):
            gather_desc(t, p).wait()

            @pl.when(t + 1 < _NCHUNK)
            def _():
                gather_desc(t + 1, 1 - p).start()

            prev = None
            for r2 in range(0, 128, 2):
                vs = [gbuf[p][r2 + (kk // 4), pl.ds(16 * (kk % 4), 16)]
                      for kk in range(8)]
                if prev is not None:
                    pr2, pvs = prev
                    for kk in range(8):
                        plsc.store_scatter(
                            tbuf[p],
                            [tdvecs[kk % 4], ddvecs[kk % 4],
                             jnp.full((16,), pr2 + (kk // 4), jnp.int32)],
                            pvs[kk])
                prev = (r2, vs)
            pr2, pvs = prev
            for kk in range(8):
                plsc.store_scatter(
                    tbuf[p],
                    [tdvecs[kk % 4], ddvecs[kk % 4],
                     jnp.full((16,), pr2 + (kk // 4), jnp.int32)],
                    pvs[kk])

            @pl.when(t >= 2)
            def _():
                for d in store_descs(t - 2, p):
                    d.wait()

            for d in store_descs(t, p):
                d.start()

        @pl.when(p == 0)
        def _():
            run(0)

        @pl.when(p == 1)
        def _():
            run(1)

        return carry

    lax.fori_loop(0, _NCHUNK, step, 0)
    for d in store_descs(_NCHUNK - 2, 0):
        d.wait()
    for d in store_descs(_NCHUNK - 1, 1):
        d.wait()


def kernel(x, table):
    xt2 = (x.T * 2).astype(jnp.int32)
    tpad = jnp.pad(table, ((0, 0), (0, _D))).reshape(2 * _NROWS, _D)
    out5 = _embed(xt2, tpad)
    return out5.transpose(2, 4, 0, 1, 3).reshape(_BATCH, _SEQ, _D)
